# baseline (speedup 1.0000x reference)
import functools

import jax
import jax.numpy as jnp
from jax import lax
from jax.experimental import pallas as pl
from jax.experimental.pallas import tpu as pltpu
from jax.experimental.pallas import tpu_sc as plsc

N_NODES = 10000
N_EDGES = 320000
D_NODE = 128
D_EDGE = 16
D_HID = 128

NC = 2
NS = 16
NW = NC * NS
E_PER_W = N_EDGES // NW
K = 80
CHUNKS = E_PER_W // K
N_PAD = 10240
ROWS_PER_TILE = N_PAD // NS


def _leaky(x):
    return jnp.where(x >= 0, x, 0.01 * x)


def _edge_proj_body(e_ref, w1_ref, b1_ref, w2_ref, b2_ref, o_ref):
    h = jnp.dot(e_ref[...], w1_ref[...], preferred_element_type=jnp.float32)
    h = _leaky(h + b1_ref[...])
    h = jnp.dot(h, w2_ref[...], preferred_element_type=jnp.float32)
    h = _leaky(h + b2_ref[...]).astype(jnp.bfloat16)
    h16 = jax.lax.bitcast_convert_type(h, jnp.int16)
    lo = h16[:, : D_HID // 2].astype(jnp.int32) & 0xFFFF
    hi = h16[:, D_HID // 2:].astype(jnp.int32) << 16
    o_ref[...] = lo | hi


def _edge_proj(edges, w1, b1, w2, b2):
    BLK = 8000
    return pl.pallas_call(
        _edge_proj_body,
        grid=(N_EDGES // BLK,),
        in_specs=[
            pl.BlockSpec((BLK, D_EDGE), lambda i: (i, 0)),
            pl.BlockSpec((D_EDGE, D_HID), lambda i: (0, 0)),
            pl.BlockSpec((1, D_HID), lambda i: (0, 0)),
            pl.BlockSpec((D_HID, D_HID), lambda i: (0, 0)),
            pl.BlockSpec((1, D_HID), lambda i: (0, 0)),
        ],
        out_specs=pl.BlockSpec((BLK, D_HID // 2), lambda i: (i, 0)),
        out_shape=jax.ShapeDtypeStruct((N_EDGES, D_HID // 2), jnp.int32),
    )(edges, w1, b1.reshape(1, D_HID), w2, b2.reshape(1, D_HID))


def _node_proj_body(n_ref, w_ref, o_ref):
    o_ref[...] = jnp.dot(n_ref[...], w_ref[...],
                         preferred_element_type=jnp.float32)


def _node_proj(nodes, w):
    BLK = 2000
    return pl.pallas_call(
        _node_proj_body,
        grid=(N_NODES // BLK,),
        in_specs=[
            pl.BlockSpec((BLK, D_NODE), lambda i: (i, 0)),
            pl.BlockSpec((D_NODE, D_HID), lambda i: (0, 0)),
        ],
        out_specs=pl.BlockSpec((BLK, D_HID), lambda i: (i, 0)),
        out_shape=jax.ShapeDtypeStruct((N_NODES, D_HID), jnp.float32),
    )(nodes, w)


def _combine_body(a_ref, b_ref, o_ref):
    o_ref[...] = a_ref[...] + b_ref[...]


def _combine(a, b):
    BLK = 2000
    return pl.pallas_call(
        _combine_body,
        grid=(N_NODES // BLK,),
        in_specs=[
            pl.BlockSpec((BLK, D_NODE), lambda i: (i, 0)),
            pl.BlockSpec((BLK, D_NODE), lambda i: (i, 0)),
        ],
        out_specs=pl.BlockSpec((BLK, D_NODE), lambda i: (i, 0)),
        out_shape=jax.ShapeDtypeStruct((N_NODES, D_NODE), jnp.float32),
    )(a, b)


@functools.partial(
    pl.kernel,
    out_type=(
        jax.ShapeDtypeStruct((N_PAD, D_NODE), jnp.float32),
        jax.ShapeDtypeStruct((N_PAD, D_NODE), jnp.float32),
    ),
    mesh=plsc.VectorSubcoreMesh(core_axis_name="c", subcore_axis_name="s"),
    scratch_types=[
        pltpu.VMEM((K,), jnp.int32),
        pltpu.VMEM((K,), jnp.int32),
        pltpu.VMEM((K,), jnp.int32),
        pltpu.VMEM((K,), jnp.int32),
        pltpu.VMEM((K,), jnp.int32),
        pltpu.VMEM((K,), jnp.int32),
        pltpu.VMEM((K,), jnp.int32),
        pltpu.VMEM((K,), jnp.int32),
        pltpu.VMEM((2, K, D_NODE), jnp.float32),
        pltpu.VMEM((2, K, D_NODE // 2), jnp.int32),
        pltpu.VMEM_SHARED((N_PAD, D_NODE), jnp.float32),
        pltpu.SemaphoreType.DMA,
        pltpu.SemaphoreType.DMA,
        pltpu.SemaphoreType.DMA,
        pltpu.SemaphoreType.DMA,
        pltpu.SemaphoreType.DMA,
        pltpu.SemaphoreType.DMA,
        pltpu.SemaphoreType.DMA,
        pltpu.SemaphoreType.DMA,
    ],
)
def _sc_gather_scatter(m_hbm, e_hbm, idx_hbm, seg_hbm, z_hbm,
                       out0_hbm, out1_hbm,
                       idx_r0, idx_r1, idx_r2, idx_r3,
                       seg_r0, seg_r1, seg_r2, seg_r3,
                       rows_v, e_v, acc,
                       gsem0, gsem1, esem0, esem1,
                       isem0, isem1, isem2, isem3):
    c = lax.axis_index("c")
    s = lax.axis_index("s")
    wid = s * NC + c
    gsems = (gsem0, gsem1)
    esems = (esem0, esem1)
    isems = (isem0, isem1, isem2, isem3)
    idx_r = (idx_r0, idx_r1, idx_r2, idx_r3)
    seg_r = (seg_r0, seg_r1, seg_r2, seg_r3)

    pltpu.sync_copy(z_hbm, acc.at[pl.ds(s * ROWS_PER_TILE, ROWS_PER_TILE)])

    def _idx_src(ci):
        base = pl.multiple_of(wid * E_PER_W + ci * K, 16)
        return idx_hbm.at[pl.ds(base, K)], seg_hbm.at[pl.ds(base, K)]

    def _e_src(ci):
        base = pl.multiple_of(wid * E_PER_W + ci * K, 16)
        return e_hbm.at[pl.ds(base, K)]

    def _issue_idx(ci, slot):
        isrc, ssrc = _idx_src(ci)
        pltpu.async_copy(isrc, idx_r[slot], isems[slot])
        pltpu.async_copy(ssrc, seg_r[slot], isems[slot])

    def _wait_idx(ci, slot):
        isrc, ssrc = _idx_src(ci)
        pltpu.make_async_copy(isrc, idx_r[slot], isems[slot]).wait()
        pltpu.make_async_copy(ssrc, seg_r[slot], isems[slot]).wait()

    def _issue_data(ci, slot, b):
        pltpu.async_copy(m_hbm.at[idx_r[slot]], rows_v.at[b], gsems[b])
        pltpu.async_copy(_e_src(ci), e_v.at[b], esems[b])

    def _wait_data(ci, slot, b):
        pltpu.make_async_copy(m_hbm.at[idx_r[slot]], rows_v.at[b],
                              gsems[b]).wait()
        pltpu.make_async_copy(_e_src(ci), e_v.at[b], esems[b]).wait()

    def _process(ci, slot, b):
        _wait_data(ci, slot, b)

        def row_body(r, carry2):
            for j in range(D_NODE // 32):
                ew = e_v[b, r, pl.ds(j * 16, 16)]
                e0 = jax.lax.bitcast_convert_type(ew << 16, jnp.float32)
                e1 = jax.lax.bitcast_convert_type(ew & jnp.int32(-65536),
                                                  jnp.float32)
                sl0 = pl.ds(j * 32, 16)
                sl1 = pl.ds(j * 32 + 16, 16)
                rows_v[b, r, sl0] = rows_v[b, r, sl0] * e0
                rows_v[b, r, sl1] = rows_v[b, r, sl1] * e1
            return carry2

        lax.fori_loop(0, K, row_body, 0)
        pltpu.sync_copy(rows_v.at[b], acc.at[seg_r[slot]], add=True)

    for ci in range(4):
        _issue_idx(ci, ci)
    for ci in range(2):
        _wait_idx(ci, ci)
        _issue_data(ci, ci, ci)
    plsc.subcore_barrier()

    @pl.loop(0, CHUNKS - 1, step=4)
    def _main(i):
        for b in range(4):
            ci = i + b
            _process(ci, b, b % 2)

            @pl.when(ci + 2 < CHUNKS)
            def _():
                _wait_idx(ci + 2, (b + 2) % 4)
                _issue_data(ci + 2, (b + 2) % 4, b % 2)

            @pl.when(ci + 4 < CHUNKS)
            def _():
                _issue_idx(ci + 4, b)

    _process(CHUNKS - 1, 0, 0)
    plsc.subcore_barrier()

    row0 = s * ROWS_PER_TILE
    acc_slice = acc.at[pl.ds(row0, ROWS_PER_TILE)]

    @pl.when(c == 0)
    def _():
        pltpu.sync_copy(acc_slice, out0_hbm.at[pl.ds(row0, ROWS_PER_TILE)])

    @pl.when(c == 1)
    def _():
        pltpu.sync_copy(acc_slice, out1_hbm.at[pl.ds(row0, ROWS_PER_TILE)])


def kernel(nodes, edges, segmentation_index, index, W_node, W_e1, b_e1, W_e2,
           b_e2):
    idx = index.astype(jnp.int32)
    seg = segmentation_index.astype(jnp.int32)
    p = jnp.arange(D_HID)
    q = p % 64
    half = p // 64
    perm = 32 * (q // 16) + 16 * half + (q % 16)
    e = _edge_proj(edges, W_e1, b_e1, W_e2[:, perm], b_e2[perm])
    m = _node_proj(nodes, W_node)
    z = jnp.zeros((ROWS_PER_TILE, D_NODE), jnp.float32)
    p0, p1 = _sc_gather_scatter(m, e, idx, seg, z)
    return _combine(p0[:N_NODES], p1[:N_NODES])


# speedup vs baseline: 1.0614x; 1.0614x over previous
import functools

import jax
import jax.numpy as jnp
from jax import lax
from jax.experimental import pallas as pl
from jax.experimental.pallas import tpu as pltpu
from jax.experimental.pallas import tpu_sc as plsc

N_NODES = 10000
N_EDGES = 320000
D_NODE = 128
D_EDGE = 16
D_HID = 128

NC = 2
NS = 16
NW = NC * NS
E_PER_W = N_EDGES // NW
K = 80
CHUNKS = E_PER_W // K
N_PAD = 10240
ROWS_PER_TILE = N_PAD // NS


def _leaky(x):
    return jnp.where(x >= 0, x, 0.01 * x)


def _rne_pack(h):
    u = jax.lax.bitcast_convert_type(h, jnp.int32)
    r = (u + 0x7FFF + ((u >> 16) & 1)) >> 16
    lo = r[:, : D_HID // 2] & 0xFFFF
    hi = r[:, D_HID // 2:] << 16
    return lo | hi


def _edge_proj_body(e_ref, w1_ref, b1_ref, w2_ref, b2_ref, o_ref):
    h = jnp.dot(e_ref[...], w1_ref[...], preferred_element_type=jnp.float32)
    h = _leaky(h + b1_ref[...])
    h = jnp.dot(h, w2_ref[...], preferred_element_type=jnp.float32)
    h = _leaky(h + b2_ref[...])
    half = h.shape[0] // 2
    o_ref[...] = jnp.concatenate(
        [_rne_pack(h[:half]), _rne_pack(h[half:])], axis=1)


def _edge_proj(edges, w1, b1, w2, b2):
    BLK = N_EDGES // NW
    return pl.pallas_call(
        _edge_proj_body,
        grid=(N_EDGES // BLK,),
        in_specs=[
            pl.BlockSpec((BLK, D_EDGE), lambda i: (i, 0)),
            pl.BlockSpec((D_EDGE, D_HID), lambda i: (0, 0)),
            pl.BlockSpec((1, D_HID), lambda i: (0, 0)),
            pl.BlockSpec((D_HID, D_HID), lambda i: (0, 0)),
            pl.BlockSpec((1, D_HID), lambda i: (0, 0)),
        ],
        out_specs=pl.BlockSpec((BLK // 2, D_HID), lambda i: (i, 0)),
        out_shape=jax.ShapeDtypeStruct((N_EDGES // 2, D_HID), jnp.int32),
    )(edges, w1, b1.reshape(1, D_HID), w2, b2.reshape(1, D_HID))


def _node_proj_body(n_ref, w_ref, o_ref):
    o_ref[...] = jnp.dot(n_ref[...], w_ref[...],
                         preferred_element_type=jnp.float32)


def _node_proj(nodes, w):
    BLK = 2000
    return pl.pallas_call(
        _node_proj_body,
        grid=(N_NODES // BLK,),
        in_specs=[
            pl.BlockSpec((BLK, D_NODE), lambda i: (i, 0)),
            pl.BlockSpec((D_NODE, D_HID), lambda i: (0, 0)),
        ],
        out_specs=pl.BlockSpec((BLK, D_HID), lambda i: (i, 0)),
        out_shape=jax.ShapeDtypeStruct((N_NODES, D_HID), jnp.float32),
    )(nodes, w)


def _combine_body(a_ref, b_ref, o_ref):
    o_ref[...] = a_ref[...] + b_ref[...]


def _combine(a, b):
    BLK = 2000
    return pl.pallas_call(
        _combine_body,
        grid=(N_NODES // BLK,),
        in_specs=[
            pl.BlockSpec((BLK, D_NODE), lambda i: (i, 0)),
            pl.BlockSpec((BLK, D_NODE), lambda i: (i, 0)),
        ],
        out_specs=pl.BlockSpec((BLK, D_NODE), lambda i: (i, 0)),
        out_shape=jax.ShapeDtypeStruct((N_NODES, D_NODE), jnp.float32),
    )(a, b)


@functools.partial(
    pl.kernel,
    out_type=(
        jax.ShapeDtypeStruct((N_PAD, D_NODE), jnp.float32),
        jax.ShapeDtypeStruct((N_PAD, D_NODE), jnp.float32),
    ),
    mesh=plsc.VectorSubcoreMesh(core_axis_name="c", subcore_axis_name="s"),
    scratch_types=[
        pltpu.VMEM((K,), jnp.int32),
        pltpu.VMEM((K,), jnp.int32),
        pltpu.VMEM((K,), jnp.int32),
        pltpu.VMEM((K,), jnp.int32),
        pltpu.VMEM((K,), jnp.int32),
        pltpu.VMEM((K,), jnp.int32),
        pltpu.VMEM((K,), jnp.int32),
        pltpu.VMEM((K,), jnp.int32),
        pltpu.VMEM((2, K, D_NODE), jnp.float32),
        pltpu.VMEM((2, K // 2, D_NODE), jnp.int32),
        pltpu.VMEM_SHARED((N_PAD, D_NODE), jnp.float32),
        pltpu.SemaphoreType.DMA,
        pltpu.SemaphoreType.DMA,
        pltpu.SemaphoreType.DMA,
        pltpu.SemaphoreType.DMA,
        pltpu.SemaphoreType.DMA,
        pltpu.SemaphoreType.DMA,
        pltpu.SemaphoreType.DMA,
        pltpu.SemaphoreType.DMA,
    ],
)
def _sc_gather_scatter(m_hbm, e_hbm, idx_hbm, seg_hbm, z_hbm,
                       out0_hbm, out1_hbm,
                       idx_r0, idx_r1, idx_r2, idx_r3,
                       seg_r0, seg_r1, seg_r2, seg_r3,
                       rows_v, e_v, acc,
                       gsem0, gsem1, esem0, esem1,
                       isem0, isem1, isem2, isem3):
    c = lax.axis_index("c")
    s = lax.axis_index("s")
    wid = s * NC + c
    gsems = (gsem0, gsem1)
    esems = (esem0, esem1)
    isems = (isem0, isem1, isem2, isem3)
    idx_r = (idx_r0, idx_r1, idx_r2, idx_r3)
    seg_r = (seg_r0, seg_r1, seg_r2, seg_r3)

    pltpu.sync_copy(z_hbm, acc.at[pl.ds(s * ROWS_PER_TILE, ROWS_PER_TILE)])

    KH = K // 2

    def _idx_src(ci):
        lo = pl.multiple_of(wid * E_PER_W + ci * KH, 8)
        hi = pl.multiple_of(wid * E_PER_W + E_PER_W // 2 + ci * KH, 8)
        return (idx_hbm.at[pl.ds(lo, KH)], idx_hbm.at[pl.ds(hi, KH)],
                seg_hbm.at[pl.ds(lo, KH)], seg_hbm.at[pl.ds(hi, KH)])

    def _e_src(ci):
        base = pl.multiple_of(wid * (E_PER_W // 2) + ci * KH, 8)
        return e_hbm.at[pl.ds(base, KH)]

    def _issue_idx(ci, slot):
        il, ih, sl_, sh = _idx_src(ci)
        pltpu.async_copy(il, idx_r[slot].at[pl.ds(0, KH)], isems[slot])
        pltpu.async_copy(ih, idx_r[slot].at[pl.ds(KH, KH)], isems[slot])
        pltpu.async_copy(sl_, seg_r[slot].at[pl.ds(0, KH)], isems[slot])
        pltpu.async_copy(sh, seg_r[slot].at[pl.ds(KH, KH)], isems[slot])

    def _wait_idx(ci, slot):
        il, ih, sl_, sh = _idx_src(ci)
        pltpu.make_async_copy(il, idx_r[slot].at[pl.ds(0, KH)],
                              isems[slot]).wait()
        pltpu.make_async_copy(ih, idx_r[slot].at[pl.ds(KH, KH)],
                              isems[slot]).wait()
        pltpu.make_async_copy(sl_, seg_r[slot].at[pl.ds(0, KH)],
                              isems[slot]).wait()
        pltpu.make_async_copy(sh, seg_r[slot].at[pl.ds(KH, KH)],
                              isems[slot]).wait()

    def _issue_data(ci, slot, b):
        pltpu.async_copy(m_hbm.at[idx_r[slot]], rows_v.at[b], gsems[b])
        pltpu.async_copy(_e_src(ci), e_v.at[b], esems[b])

    def _wait_data(ci, slot, b):
        pltpu.make_async_copy(m_hbm.at[idx_r[slot]], rows_v.at[b],
                              gsems[b]).wait()
        pltpu.make_async_copy(_e_src(ci), e_v.at[b], esems[b]).wait()

    def _process(ci, slot, b):
        _wait_data(ci, slot, b)

        def row_body(r, carry2):
            for half in range(2):
                ro = half * (K // 2)
                co = half * (D_NODE // 2)
                for j in range(D_NODE // 32):
                    ew = e_v[b, r, pl.ds(co + j * 16, 16)]
                    e0 = jax.lax.bitcast_convert_type(ew << 16, jnp.float32)
                    e1 = jax.lax.bitcast_convert_type(
                        ew & jnp.int32(-65536), jnp.float32)
                    sl0 = pl.ds(j * 32, 16)
                    sl1 = pl.ds(j * 32 + 16, 16)
                    rows_v[b, ro + r, sl0] = rows_v[b, ro + r, sl0] * e0
                    rows_v[b, ro + r, sl1] = rows_v[b, ro + r, sl1] * e1
            return carry2

        lax.fori_loop(0, K // 2, row_body, 0)
        pltpu.sync_copy(rows_v.at[b], acc.at[seg_r[slot]], add=True)

    for ci in range(4):
        _issue_idx(ci, ci)
    for ci in range(2):
        _wait_idx(ci, ci)
        _issue_data(ci, ci, ci)
    plsc.subcore_barrier()

    @pl.loop(0, CHUNKS - 1, step=4)
    def _main(i):
        for b in range(4):
            ci = i + b
            _process(ci, b, b % 2)

            @pl.when(ci + 2 < CHUNKS)
            def _():
                _wait_idx(ci + 2, (b + 2) % 4)
                _issue_data(ci + 2, (b + 2) % 4, b % 2)

            @pl.when(ci + 4 < CHUNKS)
            def _():
                _issue_idx(ci + 4, b)

    _process(CHUNKS - 1, 0, 0)
    plsc.subcore_barrier()

    row0 = s * ROWS_PER_TILE
    acc_slice = acc.at[pl.ds(row0, ROWS_PER_TILE)]

    @pl.when(c == 0)
    def _():
        pltpu.sync_copy(acc_slice, out0_hbm.at[pl.ds(row0, ROWS_PER_TILE)])

    @pl.when(c == 1)
    def _():
        pltpu.sync_copy(acc_slice, out1_hbm.at[pl.ds(row0, ROWS_PER_TILE)])


def kernel(nodes, edges, segmentation_index, index, W_node, W_e1, b_e1, W_e2,
           b_e2):
    idx = index.astype(jnp.int32)
    seg = segmentation_index.astype(jnp.int32)
    p = jnp.arange(D_HID)
    q = p % 64
    half = p // 64
    perm = 32 * (q // 16) + 16 * half + (q % 16)
    e = _edge_proj(edges, W_e1, b_e1, W_e2[:, perm], b_e2[perm])
    m = _node_proj(nodes, W_node)
    z = jnp.zeros((ROWS_PER_TILE, D_NODE), jnp.float32)
    p0, p1 = _sc_gather_scatter(m, e, idx, seg, z)
    return _combine(p0[:N_NODES], p1[:N_NODES])


# speedup vs baseline: 1.2937x; 1.2188x over previous
import functools

import jax
import jax.numpy as jnp
from jax import lax
from jax.experimental import pallas as pl
from jax.experimental.pallas import tpu as pltpu
from jax.experimental.pallas import tpu_sc as plsc

N_NODES = 10000
N_EDGES = 320000
D_NODE = 128
D_EDGE = 16
D_HID = 128

NC = 2
NS = 16
NW = NC * NS
E_PER_W = N_EDGES // NW
K = 80
CHUNKS = E_PER_W // K
E_BLK = 12800
N_PAD = 10240
ROWS_PER_TILE = N_PAD // NS


def _leaky(x):
    return jnp.where(x >= 0, x, 0.01 * x)


def _rne_pack(h):
    u = jax.lax.bitcast_convert_type(h, jnp.int32)
    r = (u + 0x7FFF + ((u >> 16) & 1)) >> 16
    lo = r[:, : D_HID // 2] & 0xFFFF
    hi = r[:, D_HID // 2:] << 16
    return lo | hi


def _edge_proj_body(et_ref, w1_ref, b1_ref, w2_ref, b2_ref, o_ref):
    h = jax.lax.dot_general(et_ref[...], w1_ref[...],
                            dimension_numbers=(((0,), (0,)), ((), ())),
                            preferred_element_type=jnp.float32)
    h = _leaky(h + b1_ref[...])
    h = jnp.dot(h, w2_ref[...], preferred_element_type=jnp.float32)
    h = _leaky(h + b2_ref[...])
    half = h.shape[0] // 2
    o_ref[:, : D_HID // 2] = _rne_pack(h[:half])
    o_ref[:, D_HID // 2:] = _rne_pack(h[half:])


def _edge_proj(edges_t, w1, b1, w2, b2):
    BLK = E_BLK
    return pl.pallas_call(
        _edge_proj_body,
        grid=(N_EDGES // BLK,),
        in_specs=[
            pl.BlockSpec((D_EDGE, BLK), lambda i: (0, i)),
            pl.BlockSpec((D_EDGE, D_HID), lambda i: (0, 0)),
            pl.BlockSpec((1, D_HID), lambda i: (0, 0)),
            pl.BlockSpec((D_HID, D_HID), lambda i: (0, 0)),
            pl.BlockSpec((1, D_HID), lambda i: (0, 0)),
        ],
        out_specs=pl.BlockSpec((BLK // 2, D_HID), lambda i: (i, 0)),
        out_shape=jax.ShapeDtypeStruct((N_EDGES // 2, D_HID), jnp.int32),
    )(edges_t, w1, b1.reshape(1, D_HID), w2, b2.reshape(1, D_HID))


def _node_proj_body(n_ref, w_ref, o_ref):
    o_ref[...] = jnp.dot(n_ref[...], w_ref[...],
                         preferred_element_type=jnp.float32)


def _node_proj(nodes, w):
    BLK = 2000
    return pl.pallas_call(
        _node_proj_body,
        grid=(N_NODES // BLK,),
        in_specs=[
            pl.BlockSpec((BLK, D_NODE), lambda i: (i, 0)),
            pl.BlockSpec((D_NODE, D_HID), lambda i: (0, 0)),
        ],
        out_specs=pl.BlockSpec((BLK, D_HID), lambda i: (i, 0)),
        out_shape=jax.ShapeDtypeStruct((N_NODES, D_HID), jnp.float32),
    )(nodes, w)


def _combine_body(a_ref, b_ref, o_ref):
    o_ref[...] = a_ref[...] + b_ref[...]


def _combine(a, b):
    BLK = 2000
    return pl.pallas_call(
        _combine_body,
        grid=(N_NODES // BLK,),
        in_specs=[
            pl.BlockSpec((BLK, D_NODE), lambda i: (i, 0)),
            pl.BlockSpec((BLK, D_NODE), lambda i: (i, 0)),
        ],
        out_specs=pl.BlockSpec((BLK, D_NODE), lambda i: (i, 0)),
        out_shape=jax.ShapeDtypeStruct((N_NODES, D_NODE), jnp.float32),
    )(a, b)


@functools.partial(
    pl.kernel,
    out_type=(
        jax.ShapeDtypeStruct((N_PAD, D_NODE), jnp.float32),
        jax.ShapeDtypeStruct((N_PAD, D_NODE), jnp.float32),
    ),
    mesh=plsc.VectorSubcoreMesh(core_axis_name="c", subcore_axis_name="s"),
    scratch_types=[
        pltpu.VMEM((K,), jnp.int32),
        pltpu.VMEM((K,), jnp.int32),
        pltpu.VMEM((K,), jnp.int32),
        pltpu.VMEM((K,), jnp.int32),
        pltpu.VMEM((K,), jnp.int32),
        pltpu.VMEM((K,), jnp.int32),
        pltpu.VMEM((K,), jnp.int32),
        pltpu.VMEM((K,), jnp.int32),
        pltpu.VMEM((2, K, D_NODE), jnp.float32),
        pltpu.VMEM((2, K // 2, D_NODE), jnp.int32),
        pltpu.VMEM_SHARED((N_PAD, D_NODE), jnp.float32),
        pltpu.SemaphoreType.DMA,
        pltpu.SemaphoreType.DMA,
        pltpu.SemaphoreType.DMA,
        pltpu.SemaphoreType.DMA,
        pltpu.SemaphoreType.DMA,
        pltpu.SemaphoreType.DMA,
        pltpu.SemaphoreType.DMA,
        pltpu.SemaphoreType.DMA,
    ],
)
def _sc_gather_scatter(m_hbm, e_hbm, idx_hbm, seg_hbm, z_hbm,
                       out0_hbm, out1_hbm,
                       idx_r0, idx_r1, idx_r2, idx_r3,
                       seg_r0, seg_r1, seg_r2, seg_r3,
                       rows_v, e_v, acc,
                       gsem0, gsem1, esem0, esem1,
                       isem0, isem1, isem2, isem3):
    c = lax.axis_index("c")
    s = lax.axis_index("s")
    wid = s * NC + c
    gsems = (gsem0, gsem1)
    esems = (esem0, esem1)
    isems = (isem0, isem1, isem2, isem3)
    idx_r = (idx_r0, idx_r1, idx_r2, idx_r3)
    seg_r = (seg_r0, seg_r1, seg_r2, seg_r3)

    pltpu.sync_copy(z_hbm, acc.at[pl.ds(s * ROWS_PER_TILE, ROWS_PER_TILE)])

    KH = K // 2

    def _bases(ci):
        g0 = pl.multiple_of(wid * (E_PER_W // 2) + ci * KH, 8)
        k = g0 // (E_BLK // 2)
        r = g0 - k * (E_BLK // 2)
        lo = pl.multiple_of(k * E_BLK + r, 8)
        hi = pl.multiple_of(lo + E_BLK // 2, 8)
        return g0, lo, hi

    def _idx_src(ci):
        _, lo, hi = _bases(ci)
        return (idx_hbm.at[pl.ds(lo, KH)], idx_hbm.at[pl.ds(hi, KH)],
                seg_hbm.at[pl.ds(lo, KH)], seg_hbm.at[pl.ds(hi, KH)])

    def _e_src(ci):
        g0, _, _ = _bases(ci)
        return e_hbm.at[pl.ds(g0, KH)]

    def _issue_idx(ci, slot):
        il, ih, sl_, sh = _idx_src(ci)
        pltpu.async_copy(il, idx_r[slot].at[pl.ds(0, KH)], isems[slot])
        pltpu.async_copy(ih, idx_r[slot].at[pl.ds(KH, KH)], isems[slot])
        pltpu.async_copy(sl_, seg_r[slot].at[pl.ds(0, KH)], isems[slot])
        pltpu.async_copy(sh, seg_r[slot].at[pl.ds(KH, KH)], isems[slot])

    def _wait_idx(ci, slot):
        il, ih, sl_, sh = _idx_src(ci)
        pltpu.make_async_copy(il, idx_r[slot].at[pl.ds(0, KH)],
                              isems[slot]).wait()
        pltpu.make_async_copy(ih, idx_r[slot].at[pl.ds(KH, KH)],
                              isems[slot]).wait()
        pltpu.make_async_copy(sl_, seg_r[slot].at[pl.ds(0, KH)],
                              isems[slot]).wait()
        pltpu.make_async_copy(sh, seg_r[slot].at[pl.ds(KH, KH)],
                              isems[slot]).wait()

    def _issue_data(ci, slot, b):
        pltpu.async_copy(m_hbm.at[idx_r[slot]], rows_v.at[b], gsems[b])
        pltpu.async_copy(_e_src(ci), e_v.at[b], esems[b])

    def _wait_data(ci, slot, b):
        pltpu.make_async_copy(m_hbm.at[idx_r[slot]], rows_v.at[b],
                              gsems[b]).wait()
        pltpu.make_async_copy(_e_src(ci), e_v.at[b], esems[b]).wait()

    def _process(ci, slot, b):
        _wait_data(ci, slot, b)

        def row_body(r, carry2):
            for half in range(2):
                ro = half * (K // 2)
                co = half * (D_NODE // 2)
                for j in range(D_NODE // 32):
                    ew = e_v[b, r, pl.ds(co + j * 16, 16)]
                    e0 = jax.lax.bitcast_convert_type(ew << 16, jnp.float32)
                    e1 = jax.lax.bitcast_convert_type(
                        ew & jnp.int32(-65536), jnp.float32)
                    sl0 = pl.ds(j * 32, 16)
                    sl1 = pl.ds(j * 32 + 16, 16)
                    rows_v[b, ro + r, sl0] = rows_v[b, ro + r, sl0] * e0
                    rows_v[b, ro + r, sl1] = rows_v[b, ro + r, sl1] * e1
            return carry2

        lax.fori_loop(0, K // 2, row_body, 0)
        pltpu.sync_copy(rows_v.at[b], acc.at[seg_r[slot]], add=True)

    for ci in range(4):
        _issue_idx(ci, ci)
    for ci in range(2):
        _wait_idx(ci, ci)
        _issue_data(ci, ci, ci)
    plsc.subcore_barrier()

    @pl.loop(0, CHUNKS - 1, step=4)
    def _main(i):
        for b in range(4):
            ci = i + b
            _process(ci, b, b % 2)

            @pl.when(ci + 2 < CHUNKS)
            def _():
                _wait_idx(ci + 2, (b + 2) % 4)
                _issue_data(ci + 2, (b + 2) % 4, b % 2)

            @pl.when(ci + 4 < CHUNKS)
            def _():
                _issue_idx(ci + 4, b)

    _process(CHUNKS - 1, 0, 0)
    plsc.subcore_barrier()

    row0 = s * ROWS_PER_TILE
    acc_slice = acc.at[pl.ds(row0, ROWS_PER_TILE)]

    @pl.when(c == 0)
    def _():
        pltpu.sync_copy(acc_slice, out0_hbm.at[pl.ds(row0, ROWS_PER_TILE)])

    @pl.when(c == 1)
    def _():
        pltpu.sync_copy(acc_slice, out1_hbm.at[pl.ds(row0, ROWS_PER_TILE)])


def kernel(nodes, edges, segmentation_index, index, W_node, W_e1, b_e1, W_e2,
           b_e2):
    idx = index.astype(jnp.int32)
    seg = segmentation_index.astype(jnp.int32)
    p = jnp.arange(D_HID)
    q = p % 64
    half = p // 64
    perm = 32 * (q // 16) + 16 * half + (q % 16)
    e = _edge_proj(edges.T, W_e1, b_e1, W_e2[:, perm], b_e2[perm])
    m = _node_proj(nodes, W_node)
    z = jnp.zeros((ROWS_PER_TILE, D_NODE), jnp.float32)
    p0, p1 = _sc_gather_scatter(m, e, idx, seg, z)
    return _combine(p0, p1)


# speedup vs baseline: 1.3630x; 1.0536x over previous
import functools

import jax
import jax.numpy as jnp
from jax import lax
from jax.experimental import pallas as pl
from jax.experimental.pallas import tpu as pltpu
from jax.experimental.pallas import tpu_sc as plsc

N_NODES = 10000
N_EDGES = 320000
D_NODE = 128
D_EDGE = 16
D_HID = 128

NC = 2
NS = 16
NW = NC * NS
E_PER_W = N_EDGES // NW
K = 80
CHUNKS = E_PER_W // K
E_BLK = 12800
N_PAD = 10240
ROWS_PER_TILE = N_PAD // NS


def _leaky(x):
    return jnp.maximum(x, 0.01 * x)


def _rne_pack(h):
    u = jax.lax.bitcast_convert_type(h, jnp.int32)
    r = jax.lax.shift_right_logical(u + 0x8000, 16)
    lo = r[:, : D_HID // 2]
    hi = r[:, D_HID // 2:] << 16
    return lo | hi


def _edge_proj_body(et_ref, w1_ref, b1_ref, w2_ref, b2_ref, o_ref):
    h = jax.lax.dot_general(et_ref[...], w1_ref[...],
                            dimension_numbers=(((0,), (0,)), ((), ())),
                            preferred_element_type=jnp.float32)
    h = _leaky(h + b1_ref[...])
    h = jnp.dot(h, w2_ref[...], preferred_element_type=jnp.float32)
    h = _leaky(h + b2_ref[...])
    half = h.shape[0] // 2
    o_ref[:, : D_HID // 2] = _rne_pack(h[:half])
    o_ref[:, D_HID // 2:] = _rne_pack(h[half:])


def _edge_proj(edges_t, w1, b1, w2, b2):
    BLK = E_BLK
    return pl.pallas_call(
        _edge_proj_body,
        grid=(N_EDGES // BLK,),
        in_specs=[
            pl.BlockSpec((D_EDGE, BLK), lambda i: (0, i)),
            pl.BlockSpec((D_EDGE, D_HID), lambda i: (0, 0)),
            pl.BlockSpec((1, D_HID), lambda i: (0, 0)),
            pl.BlockSpec((D_HID, D_HID), lambda i: (0, 0)),
            pl.BlockSpec((1, D_HID), lambda i: (0, 0)),
        ],
        out_specs=pl.BlockSpec((BLK // 2, D_HID), lambda i: (i, 0)),
        out_shape=jax.ShapeDtypeStruct((N_EDGES // 2, D_HID), jnp.int32),
    )(edges_t, w1, b1.reshape(1, D_HID), w2, b2.reshape(1, D_HID))


def _node_proj_body(n_ref, w_ref, o_ref):
    o_ref[...] = jnp.dot(n_ref[...], w_ref[...],
                         preferred_element_type=jnp.float32)


def _node_proj(nodes, w):
    BLK = 2000
    return pl.pallas_call(
        _node_proj_body,
        grid=(N_NODES // BLK,),
        in_specs=[
            pl.BlockSpec((BLK, D_NODE), lambda i: (i, 0)),
            pl.BlockSpec((D_NODE, D_HID), lambda i: (0, 0)),
        ],
        out_specs=pl.BlockSpec((BLK, D_HID), lambda i: (i, 0)),
        out_shape=jax.ShapeDtypeStruct((N_NODES, D_HID), jnp.float32),
    )(nodes, w)


def _combine_body(a_ref, b_ref, o_ref):
    o_ref[...] = a_ref[...] + b_ref[...]


def _combine(a, b):
    BLK = 2000
    return pl.pallas_call(
        _combine_body,
        grid=(N_NODES // BLK,),
        in_specs=[
            pl.BlockSpec((BLK, D_NODE), lambda i: (i, 0)),
            pl.BlockSpec((BLK, D_NODE), lambda i: (i, 0)),
        ],
        out_specs=pl.BlockSpec((BLK, D_NODE), lambda i: (i, 0)),
        out_shape=jax.ShapeDtypeStruct((N_NODES, D_NODE), jnp.float32),
    )(a, b)


@functools.partial(
    pl.kernel,
    out_type=(
        jax.ShapeDtypeStruct((N_PAD, D_NODE), jnp.float32),
        jax.ShapeDtypeStruct((N_PAD, D_NODE), jnp.float32),
    ),
    mesh=plsc.VectorSubcoreMesh(core_axis_name="c", subcore_axis_name="s"),
    scratch_types=[
        pltpu.VMEM((K,), jnp.int32),
        pltpu.VMEM((K,), jnp.int32),
        pltpu.VMEM((K,), jnp.int32),
        pltpu.VMEM((K,), jnp.int32),
        pltpu.VMEM((K,), jnp.int32),
        pltpu.VMEM((K,), jnp.int32),
        pltpu.VMEM((K,), jnp.int32),
        pltpu.VMEM((K,), jnp.int32),
        pltpu.VMEM((2, K, D_NODE), jnp.float32),
        pltpu.VMEM((2, K // 2, D_NODE), jnp.int32),
        pltpu.VMEM_SHARED((N_PAD, D_NODE), jnp.float32),
        pltpu.SemaphoreType.DMA,
        pltpu.SemaphoreType.DMA,
        pltpu.SemaphoreType.DMA,
        pltpu.SemaphoreType.DMA,
        pltpu.SemaphoreType.DMA,
        pltpu.SemaphoreType.DMA,
        pltpu.SemaphoreType.DMA,
        pltpu.SemaphoreType.DMA,
    ],
)
def _sc_gather_scatter(m_hbm, e_hbm, idx_hbm, seg_hbm, z_hbm,
                       out0_hbm, out1_hbm,
                       idx_r0, idx_r1, idx_r2, idx_r3,
                       seg_r0, seg_r1, seg_r2, seg_r3,
                       rows_v, e_v, acc,
                       gsem0, gsem1, esem0, esem1,
                       isem0, isem1, isem2, isem3):
    c = lax.axis_index("c")
    s = lax.axis_index("s")
    wid = s * NC + c
    gsems = (gsem0, gsem1)
    esems = (esem0, esem1)
    isems = (isem0, isem1, isem2, isem3)
    idx_r = (idx_r0, idx_r1, idx_r2, idx_r3)
    seg_r = (seg_r0, seg_r1, seg_r2, seg_r3)

    pltpu.sync_copy(z_hbm, acc.at[pl.ds(s * ROWS_PER_TILE, ROWS_PER_TILE)])

    KH = K // 2

    def _bases(ci):
        g0 = pl.multiple_of(wid * (E_PER_W // 2) + ci * KH, 8)
        k = g0 // (E_BLK // 2)
        r = g0 - k * (E_BLK // 2)
        lo = pl.multiple_of(k * E_BLK + r, 8)
        hi = pl.multiple_of(lo + E_BLK // 2, 8)
        return g0, lo, hi

    def _idx_src(ci):
        _, lo, hi = _bases(ci)
        return (idx_hbm.at[pl.ds(lo, KH)], idx_hbm.at[pl.ds(hi, KH)],
                seg_hbm.at[pl.ds(lo, KH)], seg_hbm.at[pl.ds(hi, KH)])

    def _e_src(ci):
        g0, _, _ = _bases(ci)
        return e_hbm.at[pl.ds(g0, KH)]

    def _issue_idx(ci, slot):
        il, ih, sl_, sh = _idx_src(ci)
        pltpu.async_copy(il, idx_r[slot].at[pl.ds(0, KH)], isems[slot])
        pltpu.async_copy(ih, idx_r[slot].at[pl.ds(KH, KH)], isems[slot])
        pltpu.async_copy(sl_, seg_r[slot].at[pl.ds(0, KH)], isems[slot])
        pltpu.async_copy(sh, seg_r[slot].at[pl.ds(KH, KH)], isems[slot])

    def _wait_idx(ci, slot):
        il, ih, sl_, sh = _idx_src(ci)
        pltpu.make_async_copy(il, idx_r[slot].at[pl.ds(0, KH)],
                              isems[slot]).wait()
        pltpu.make_async_copy(ih, idx_r[slot].at[pl.ds(KH, KH)],
                              isems[slot]).wait()
        pltpu.make_async_copy(sl_, seg_r[slot].at[pl.ds(0, KH)],
                              isems[slot]).wait()
        pltpu.make_async_copy(sh, seg_r[slot].at[pl.ds(KH, KH)],
                              isems[slot]).wait()

    def _issue_data(ci, slot, b):
        pltpu.async_copy(m_hbm.at[idx_r[slot]], rows_v.at[b], gsems[b])
        pltpu.async_copy(_e_src(ci), e_v.at[b], esems[b])

    def _wait_data(ci, slot, b):
        pltpu.make_async_copy(m_hbm.at[idx_r[slot]], rows_v.at[b],
                              gsems[b]).wait()
        pltpu.make_async_copy(_e_src(ci), e_v.at[b], esems[b]).wait()

    def _process(ci, slot, b):
        _wait_data(ci, slot, b)

        def row_body(r, carry2):
            for half in range(2):
                ro = half * (K // 2)
                co = half * (D_NODE // 2)
                for j in range(D_NODE // 32):
                    ew = e_v[b, r, pl.ds(co + j * 16, 16)]
                    e0 = jax.lax.bitcast_convert_type(ew << 16, jnp.float32)
                    e1 = jax.lax.bitcast_convert_type(
                        ew & jnp.int32(-65536), jnp.float32)
                    sl0 = pl.ds(j * 32, 16)
                    sl1 = pl.ds(j * 32 + 16, 16)
                    rows_v[b, ro + r, sl0] = rows_v[b, ro + r, sl0] * e0
                    rows_v[b, ro + r, sl1] = rows_v[b, ro + r, sl1] * e1
            return carry2

        lax.fori_loop(0, K // 2, row_body, 0)
        pltpu.sync_copy(rows_v.at[b], acc.at[seg_r[slot]], add=True)

    for ci in range(4):
        _issue_idx(ci, ci)
    for ci in range(2):
        _wait_idx(ci, ci)
        _issue_data(ci, ci, ci)
    plsc.subcore_barrier()

    @pl.loop(0, CHUNKS - 1, step=4)
    def _main(i):
        for b in range(4):
            ci = i + b
            _process(ci, b, b % 2)

            @pl.when(ci + 2 < CHUNKS)
            def _():
                _wait_idx(ci + 2, (b + 2) % 4)
                _issue_data(ci + 2, (b + 2) % 4, b % 2)

            @pl.when(ci + 4 < CHUNKS)
            def _():
                _issue_idx(ci + 4, b)

    _process(CHUNKS - 1, 0, 0)
    plsc.subcore_barrier()

    row0 = s * ROWS_PER_TILE
    acc_slice = acc.at[pl.ds(row0, ROWS_PER_TILE)]

    @pl.when(c == 0)
    def _():
        pltpu.sync_copy(acc_slice, out0_hbm.at[pl.ds(row0, ROWS_PER_TILE)])

    @pl.when(c == 1)
    def _():
        pltpu.sync_copy(acc_slice, out1_hbm.at[pl.ds(row0, ROWS_PER_TILE)])


def kernel(nodes, edges, segmentation_index, index, W_node, W_e1, b_e1, W_e2,
           b_e2):
    idx = index.astype(jnp.int32)
    seg = segmentation_index.astype(jnp.int32)
    p = jnp.arange(D_HID)
    q = p % 64
    half = p // 64
    perm = 32 * (q // 16) + 16 * half + (q % 16)
    e = _edge_proj(edges.T, W_e1, b_e1, W_e2[:, perm], b_e2[perm])
    m = _node_proj(nodes, W_node)
    z = jnp.zeros((ROWS_PER_TILE, D_NODE), jnp.float32)
    p0, p1 = _sc_gather_scatter(m, e, idx, seg, z)
    return _combine(p0, p1)
